# baseline (speedup 1.0000x reference)
import functools

import jax
import jax.numpy as jnp
from jax import lax
from jax.experimental import pallas as pl
from jax.experimental.pallas import tpu as pltpu
from jax.experimental.pallas import tpu_sc as plsc

N = 10000
NP = 10240
E = 320000
EP = 327680
PAD = EP - E
EPR = EP // 128
NC = 2
NS = 16
NW = NC * NS
RT = NP // NS
CH = 64
IB = 8
BR = 1024
F32 = jnp.float32

_mesh = functools.partial(
    plsc.VectorSubcoreMesh, core_axis_name="c", subcore_axis_name="s")


def _zero_rows(buf, nrows, width, dtype=F32):
  z = jnp.zeros((16,), dtype)
  def f(i, _):
    for k in range(width // 16):
      buf[i, pl.ds(k * 16, 16)] = z
    return 0
  lax.fori_loop(0, nrows, f, 0, unroll=False)


def _deg_body(dst_hbm, out_hbm, idx_v, hist_v):
  c = lax.axis_index("c")
  s = lax.axis_index("s")
  wid = c * NS + s
  nch = EP // NW // 128
  zi = jnp.zeros((16,), jnp.int32)
  def fz(i, _):
    hist_v[pl.ds(i * 16, 16)] = zi
    return 0
  lax.fori_loop(0, NP // 16, fz, 0, unroll=False)
  pltpu.sync_copy(dst_hbm.at[pl.ds(wid * nch, nch)], idx_v)
  cal, _ = plsc.scan_count(jnp.zeros((16,), jnp.int32))
  corr = 16 - jnp.max(cal)
  def step(i, _):
    iv = idx_v[i // 8, pl.ds((i % 8) * 16, 16)]
    cnt, last = plsc.scan_count(iv)
    plsc.addupdate_scatter(hist_v, [iv], cnt + corr, mask=last)
    return 0
  lax.fori_loop(0, nch * 8, step, 0, unroll=False)
  pltpu.sync_copy(hist_v, out_hbm.at[wid])


def _sc_degree(dst2d):
  nch = EP // NW // 128
  return pl.kernel(
      _deg_body,
      out_type=jax.ShapeDtypeStruct((NW, NP), jnp.int32),
      mesh=_mesh(),
      compiler_params=pltpu.CompilerParams(needs_layout_passes=False),
      scratch_types=[
          pltpu.VMEM((nch, 128), jnp.int32),
          pltpu.VMEM((NP,), jnp.int32),
      ],
  )(dst2d)


def _spmm_body(nch, conv1, src_hbm, dst_hbm, tab_hbm, out_hbm,
               sidx, didx, b0, b1, b2, b3, acc_sh,
               g0, g1, g2, g3, s0, s1, s2, s3, isem_s, isem_d):
  c = lax.axis_index("c")
  s = lax.axis_index("s")
  if conv1:
    sbase = c * (EP // CH) + s * nch
    dbase = s * nch
  else:
    sbase = (c * NS + s) * nch
    dbase = sbase
  bufs = (b0, b1, b2, b3)
  gsems = (g0, g1, g2, g3)
  ssems = (s0, s1, s2, s3)
  _zero_rows(b0, CH, 128)
  for k in range(RT // CH):
    pltpu.sync_copy(b0, acc_sh.at[pl.ds(s * RT + k * CH, CH)])
  plsc.subcore_barrier()
  nst = nch // IB

  def load_idx(bank, t, wait):
    a = pltpu.make_async_copy(
        src_hbm.at[pl.ds(sbase + t * IB, IB)], sidx.at[pl.ds(bank * IB, IB)],
        isem_s)
    b = pltpu.make_async_copy(
        dst_hbm.at[pl.ds(dbase + t * IB, IB)], didx.at[pl.ds(bank * IB, IB)],
        isem_d)
    if wait:
      a.wait()
      b.wait()
    else:
      a.start()
      b.start()

  def gather(row, bi, wait):
    d = pltpu.make_async_copy(tab_hbm.at[sidx.at[row]], bufs[bi], gsems[bi])
    if wait:
      d.wait()
    else:
      d.start()

  load_idx(0, 0, wait=False)
  load_idx(0, 0, wait=True)
  load_idx(1, 1, wait=False)
  for q in range(3):
    gather(q, q, wait=False)

  def stage_pair(tp, _):
    for bank in (0, 1):
      t = tp * 2 + bank
      nb = 1 - bank
      for ql in range(IB):
        bq = ql % 4
        rq = bank * IB + ql
        pb = (ql - 1) % 4
        gather(rq, bq, wait=True)
        pltpu.async_copy(bufs[bq], acc_sh.at[didx.at[rq]], ssems[bq],
                         add=True)
        if ql == 0:
          @pl.when(t >= 1)
          def _():
            pltpu.make_async_copy(
                bufs[pb], acc_sh.at[didx.at[rq]], ssems[pb]).wait()
          @pl.when(jnp.logical_and(t >= 1, t + 1 < nst))
          def _():
            load_idx(nb, t + 1, wait=False)
        else:
          pltpu.make_async_copy(
              bufs[pb], acc_sh.at[didx.at[rq]], ssems[pb]).wait()
        if ql == IB - 4:
          @pl.when(t + 1 < nst)
          def _():
            load_idx(nb, t + 1, wait=True)
        if ql < IB - 3:
          gather(rq + 3, pb, wait=False)
        else:
          @pl.when(t + 1 < nst)
          def _():
            gather(nb * IB + ql + 3 - IB, pb, wait=False)
    return 0
  lax.fori_loop(0, nst // 2, stage_pair, 0, unroll=False)
  pltpu.make_async_copy(
      bufs[(IB - 1) % 4], acc_sh.at[didx.at[0]], ssems[(IB - 1) % 4]).wait()
  plsc.subcore_barrier()
  obase = c * NP + s * RT
  for k in range(RT // CH):
    pltpu.sync_copy(acc_sh.at[pl.ds(s * RT + k * CH, CH)], b0)
    pltpu.sync_copy(b0, out_hbm.at[pl.ds(obase + k * CH, CH)])


def _sc_spmm(src2d, dst2d, table, conv1):
  nch = EP // NS // CH if conv1 else EP // NW // CH
  body = functools.partial(_spmm_body, nch, conv1)
  return pl.kernel(
      body,
      out_type=jax.ShapeDtypeStruct((NC * NP, 128), F32),
      mesh=_mesh(),
      scratch_types=[
          pltpu.VMEM((2 * IB, CH), jnp.int32),
          pltpu.VMEM((2 * IB, CH), jnp.int32),
          pltpu.VMEM((CH, 128), F32),
          pltpu.VMEM((CH, 128), F32),
          pltpu.VMEM((CH, 128), F32),
          pltpu.VMEM((CH, 128), F32),
          pltpu.VMEM_SHARED((NP, 128), F32),
          pltpu.SemaphoreType.DMA,
          pltpu.SemaphoreType.DMA,
          pltpu.SemaphoreType.DMA,
          pltpu.SemaphoreType.DMA,
          pltpu.SemaphoreType.DMA,
          pltpu.SemaphoreType.DMA,
          pltpu.SemaphoreType.DMA,
          pltpu.SemaphoreType.DMA,
          pltpu.SemaphoreType.DMA,
          pltpu.SemaphoreType.DMA,
      ],
  )(src2d, dst2d, table)


def _edge_body(a_hbm, c_hbm, src_hbm, dst_hbm, y_hbm,
               a_v, c_v, si, di, y_v):
  c = lax.axis_index("c")
  s = lax.axis_index("s")
  wid = c * NS + s
  ept = E // NW
  pltpu.sync_copy(a_hbm, a_v)
  pltpu.sync_copy(c_hbm, c_v)
  pltpu.sync_copy(src_hbm.at[pl.ds(wid * ept, ept)], si)
  pltpu.sync_copy(dst_hbm.at[pl.ds(wid * ept, ept)], di)
  def step(i, _):
    iv = si[pl.ds(i * 16, 16)]
    jv = di[pl.ds(i * 16, 16)]
    av = plsc.load_gather(a_v, [iv])
    cv = plsc.load_gather(c_v, [jv])
    t = av + cv
    y_v[pl.ds(i * 16, 16)] = 1.0 / (1.0 + jnp.exp(-t))
    return 0
  lax.fori_loop(0, ept // 16, step, 0, unroll=False)
  pltpu.sync_copy(y_v, y_hbm.at[pl.ds(wid * ept, ept)])


def _sc_edge(a, cc, src1d, dst1d):
  ept = E // NW
  return pl.kernel(
      _edge_body,
      out_type=jax.ShapeDtypeStruct((E,), F32),
      mesh=_mesh(),
      compiler_params=pltpu.CompilerParams(needs_layout_passes=False),
      scratch_types=[
          pltpu.VMEM((NP,), F32),
          pltpu.VMEM((NP,), F32),
          pltpu.VMEM((ept,), jnp.int32),
          pltpu.VMEM((ept,), jnp.int32),
          pltpu.VMEM((ept,), F32),
      ],
  )(a, cc, src1d, dst1d)


def _dinv_block(degw, i):
  deg = jnp.sum(degw, axis=0).astype(F32)[:, None] + 1.0
  rows = i * BR + lax.broadcasted_iota(jnp.int32, (BR, 1), 0)
  return jnp.where(rows < N, lax.rsqrt(deg), 0.0)


def _tc1_body(x_ref, we_ref, be_ref, wg1_ref, degw_ref, o_ref):
  i = pl.program_id(0)
  h0 = jnp.maximum(x_ref[...] @ we_ref[...] + be_ref[...][None, :], 0.0)
  hw1 = h0 @ wg1_ref[...]
  dinv = _dinv_block(degw_ref[...], i)
  g1 = dinv * hw1
  o_ref[0] = g1[:, :128]
  o_ref[1] = g1[:, 128:]


def _tc2_body(s1_ref, g1_ref, degw_ref, bg1_ref, wg2_ref, o_ref):
  i = pl.program_id(0)
  dinv = _dinv_block(degw_ref[...], i)
  t = s1_ref[...] + g1_ref[...]
  pre = jnp.concatenate([t[0], t[1]], axis=1)
  h1 = jnp.maximum(dinv * pre + bg1_ref[...][None, :], 0.0)
  o_ref[...] = dinv * (h1 @ wg2_ref[...])


def _tc3_body(s2_ref, g2_ref, degw_ref, bg2_ref, wsw_ref, bsw_ref,
              wv_ref, bv_ref, ac_ref, v_ref):
  i = pl.program_id(0)
  dinv = _dinv_block(degw_ref[...], i)
  pre = s2_ref[0] + s2_ref[1] + g2_ref[...]
  h2 = jnp.maximum(dinv * pre + bg2_ref[...][None, :], 0.0)
  a = h2 @ wsw_ref[:128, :] + bsw_ref[...][None, :]
  cc = h2 @ wsw_ref[128:, :]
  vr = jax.nn.sigmoid(h2 @ wv_ref[...] + bv_ref[...][None, :])
  v = 0.9 + 0.2 * vr
  ac_ref[0] = a
  ac_ref[1] = cc
  v_ref[...] = v * v


def _row_spec(shape):
  nd = len(shape)
  return pl.BlockSpec(shape, lambda i, _n=nd: (0,) * _n)


def _tc1(x_pad, W_enc, b_enc, W_g1, degw):
  return pl.pallas_call(
      _tc1_body,
      grid=(NP // BR,),
      in_specs=[
          pl.BlockSpec((BR, 128), lambda i: (i, 0)),
          _row_spec((128, 256)),
          _row_spec((256,)),
          _row_spec((256, 256)),
          pl.BlockSpec((NW, BR), lambda i: (0, i)),
      ],
      out_specs=pl.BlockSpec((2, BR, 128), lambda i: (0, i, 0)),
      out_shape=jax.ShapeDtypeStruct((2, NP, 128), F32),
  )(x_pad, W_enc, b_enc, W_g1, degw)


def _tc2(s1, g1, degw, b_g1, W_g2):
  return pl.pallas_call(
      _tc2_body,
      grid=(NP // BR,),
      in_specs=[
          pl.BlockSpec((2, BR, 128), lambda i: (0, i, 0)),
          pl.BlockSpec((2, BR, 128), lambda i: (0, i, 0)),
          pl.BlockSpec((NW, BR), lambda i: (0, i)),
          _row_spec((256,)),
          _row_spec((256, 128)),
      ],
      out_specs=pl.BlockSpec((BR, 128), lambda i: (i, 0)),
      out_shape=jax.ShapeDtypeStruct((NP, 128), F32),
  )(s1, g1, degw, b_g1, W_g2)


def _tc3(s2, g2, degw, b_g2, W_sw, b_sw, W_v, b_v):
  return pl.pallas_call(
      _tc3_body,
      grid=(NP // BR,),
      in_specs=[
          pl.BlockSpec((2, BR, 128), lambda i: (0, i, 0)),
          pl.BlockSpec((BR, 128), lambda i: (i, 0)),
          pl.BlockSpec((NW, BR), lambda i: (0, i)),
          _row_spec((128,)),
          _row_spec((256, 1)),
          _row_spec((1,)),
          _row_spec((128, 1)),
          _row_spec((1,)),
      ],
      out_specs=[
          pl.BlockSpec((2, BR, 1), lambda i: (0, i, 0)),
          pl.BlockSpec((BR, 1), lambda i: (i, 0)),
      ],
      out_shape=[
          jax.ShapeDtypeStruct((2, NP, 1), F32),
          jax.ShapeDtypeStruct((NP, 1), F32),
      ],
  )(s2, g2, degw, b_g2, W_sw, b_sw, W_v, b_v)


def kernel(x, edge_index, W_enc, b_enc, W_g1, b_g1, W_g2, b_g2,
           W_sw, b_sw, W_v, b_v):
  src = edge_index[0]
  dst = edge_index[1]
  pad = N + (jnp.arange(PAD, dtype=jnp.int32) % (NP - N))
  srcp = jnp.concatenate([src, pad])
  dstp = jnp.concatenate([dst, pad])
  src2d = srcp.reshape(EP // CH, CH)
  dst2d = dstp.reshape(EP // CH, CH)
  dst2d_deg = dstp.reshape(EPR, 128)
  src2d_c = jnp.concatenate([src2d, src2d + NP], axis=0)
  x_pad = jnp.pad(x, ((0, NP - N), (0, 0)))

  degw = _sc_degree(dst2d_deg)
  g1 = _tc1(x_pad, W_enc, b_enc, W_g1, degw)
  s1 = _sc_spmm(src2d_c, dst2d, g1.reshape(NC * NP, 128), conv1=True)
  g2 = _tc2(s1.reshape(2, NP, 128), g1, degw, b_g1, W_g2)
  s2 = _sc_spmm(src2d, dst2d, g2, conv1=False)
  ac, v = _tc3(s2.reshape(2, NP, 128), g2, degw, b_g2, W_sw, b_sw, W_v, b_v)
  a = ac[0, :, 0]
  cc = ac[1, :, 0]
  y_warm = _sc_edge(a, cc, srcp, dstp)
  v_warm = v[:N, 0]
  return (y_warm, v_warm)
